# baseline (speedup 1.0000x reference)
import functools
import math

import jax
import jax.numpy as jnp
from jax.experimental import pallas as pl
from jax.experimental.pallas import tpu as pltpu
from jax.experimental.pallas import tpu_sc as plsc

N = 4608
IN_FEATURES = 768
EMBED_ENTRIES = 8192
EMBED_DIM = 256

BLOCK_N = 512
BLOCK_M = 1024
WSTAGE = 128
GRID_N = N // BLOCK_N
CHUNKS = EMBED_ENTRIES // BLOCK_M
GROUPS = BLOCK_M // 128
LOGK = math.log(EMBED_ENTRIES)

GATHER_WINDOW = 128


def _fold8(a):
    p = [a[:, c * 128:(c + 1) * 128] for c in range(a.shape[1] // 128)]
    while len(p) > 1:
        p = [p[k] + p[k + 1] for k in range(0, len(p) - 1, 2)] + (
            [p[-1]] if len(p) % 2 else [])
    return p[0]


def _tc_body(x_ref, w_hbm, b_ref, g_ref, idx_ref, diff_ref,
             w_vmem, wstage_ref, s_ref, t_ref, zmax_ref, zidx_ref,
             sem0, sem1):
    i = pl.program_id(0)

    @pl.when(i == 0)
    def _():
        sems = (sem0, sem1)
        nstage = EMBED_ENTRIES // WSTAGE

        def _chunk_copy(c, sem):
            return pltpu.make_async_copy(
                w_hbm.at[pl.ds(c * WSTAGE, WSTAGE), :],
                wstage_ref.at[c % 2], sem)

        _chunk_copy(0, sems[0]).start()
        for c in range(nstage):
            if c + 1 < nstage:
                _chunk_copy(c + 1, sems[(c + 1) % 2]).start()
            _chunk_copy(c, sems[c % 2]).wait()
            w_vmem[pl.ds(c * WSTAGE, WSTAGE), :] = (
                wstage_ref[c % 2].astype(jnp.bfloat16))

    xb = x_ref[...].astype(jnp.bfloat16)

    s_ref[...] = jnp.zeros_like(s_ref)
    t_ref[...] = jnp.zeros_like(t_ref)
    zmax_ref[...] = jnp.full_like(zmax_ref, -jnp.inf)
    zidx_ref[...] = jnp.zeros_like(zidx_ref)

    for c in range(CHUNKS):
        l = jax.lax.dot_general(
            xb, w_vmem[pl.ds(c * BLOCK_M, BLOCK_M), :],
            dimension_numbers=(((1,), (1,)), ((), ())),
            preferred_element_type=jnp.float32,
            precision=jax.lax.Precision.DEFAULT,
        ) + b_ref[:, c * BLOCK_M:(c + 1) * BLOCK_M]

        lb = l.astype(jnp.bfloat16)
        eb = jnp.exp(lb)
        s_ref[...] += _fold8(eb).astype(jnp.float32)
        t_ref[...] += _fold8(eb * lb).astype(jnp.float32)

        g = g_ref[:, c * BLOCK_M:(c + 1) * BLOCK_M]
        zmax = zmax_ref[...]
        zidx = zidx_ref[...]
        for q in range(GROUPS):
            zc = l[:, q * 128:(q + 1) * 128] + g[:, q * 128:(q + 1) * 128]
            upd = zc > zmax
            zmax = jnp.where(upd, zc, zmax)
            zidx = jnp.where(upd, jnp.bfloat16(c * GROUPS + q), zidx)
        zmax_ref[...] = zmax
        zidx_ref[...] = zidx

    zm = zmax_ref[...]
    lane = jax.lax.broadcasted_iota(jnp.int32, zm.shape, 1)
    col = zidx_ref[...].astype(jnp.int32) * 128 + lane
    rowmax = jnp.max(zm, axis=1, keepdims=True)
    idx = jnp.min(jnp.where(zm == rowmax, col, jnp.int32(2**30)),
                  axis=1, keepdims=True)
    idx_ref[...] = idx.reshape(1, BLOCK_N)

    s = jnp.sum(s_ref[...], axis=1, keepdims=True)
    t = jnp.sum(t_ref[...], axis=1, keepdims=True)
    drow = t / s - jnp.log(s) + LOGK
    part = (jnp.sum(drow) / N).reshape(1, 1)

    @pl.when(i == 0)
    def _():
        diff_ref[...] = part

    @pl.when(i > 0)
    def _():
        diff_ref[...] = diff_ref[...] + part


@functools.partial(jax.jit, static_argnames=("interpret",))
def _tc_part(x, W, b2d, gumbel, interpret=False):
    return pl.pallas_call(
        _tc_body,
        grid=(GRID_N,),
        in_specs=[
            pl.BlockSpec((BLOCK_N, IN_FEATURES), lambda i: (i, 0)),
            pl.BlockSpec(memory_space=pl.ANY),
            pl.BlockSpec((1, EMBED_ENTRIES), lambda i: (0, 0)),
            pl.BlockSpec((BLOCK_N, EMBED_ENTRIES), lambda i: (i, 0)),
        ],
        out_specs=[
            pl.BlockSpec((1, BLOCK_N), lambda i: (0, i)),
            pl.BlockSpec((1, 1), lambda i: (0, 0)),
        ],
        out_shape=[
            jax.ShapeDtypeStruct((1, N), jnp.int32),
            jax.ShapeDtypeStruct((1, 1), jnp.float32),
        ],
        scratch_shapes=[
            pltpu.VMEM((EMBED_ENTRIES, IN_FEATURES), jnp.bfloat16),
            pltpu.VMEM((2, WSTAGE, IN_FEATURES), jnp.float32),
            pltpu.VMEM((BLOCK_N, 128), jnp.float32),
            pltpu.VMEM((BLOCK_N, 128), jnp.float32),
            pltpu.VMEM((BLOCK_N, 128), jnp.float32),
            pltpu.VMEM((BLOCK_N, 128), jnp.bfloat16),
            pltpu.SemaphoreType.DMA,
            pltpu.SemaphoreType.DMA,
        ],
        interpret=interpret,
    )(x, W, b2d, gumbel)


def _sc_gather(embed, idx2):
    @functools.partial(
        pl.kernel,
        out_type=jax.ShapeDtypeStruct((N, EMBED_DIM), embed.dtype),
        mesh=plsc.VectorSubcoreMesh(core_axis_name="core",
                                    subcore_axis_name="subcore"),
    )
    def kern(x_hbm, i_hbm, o_hbm):
        def body(i_vmem, o_vmem):
            pltpu.sync_copy(x_hbm.at[i_vmem.at[0]], o_vmem)

        pltpu.emit_pipeline(
            body,
            grid=(N // GATHER_WINDOW,),
            in_specs=[pl.BlockSpec((1, GATHER_WINDOW),
                                   index_map=lambda i: (0, i))],
            out_specs=[pl.BlockSpec((GATHER_WINDOW, EMBED_DIM),
                                    index_map=lambda i: (i, 0))],
            core_axis_name=("core", "subcore"),
            dimension_semantics=(pltpu.PARALLEL,),
        )(i_hbm, o_hbm)

    return kern(embed, idx2)


def kernel(x, W, b, embed, gumbel):
    idx2, diff2d = _tc_part(x, W, b.reshape(1, -1), gumbel)
    quantize = _sc_gather(embed, idx2)
    return (quantize, diff2d.reshape(()), idx2.reshape(N))


# speedup vs baseline: 1.3344x; 1.3344x over previous
import functools
import math

import jax
import jax.numpy as jnp
from jax.experimental import pallas as pl
from jax.experimental.pallas import tpu as pltpu
from jax.experimental.pallas import tpu_sc as plsc

N = 4608
IN_FEATURES = 768
EMBED_ENTRIES = 8192
EMBED_DIM = 256

BLOCK_N = 256
BLOCK_M = 1024
GRID_N = N // BLOCK_N
CHUNKS = EMBED_ENTRIES // BLOCK_M
GROUPS = BLOCK_M // 128
LOGK = math.log(EMBED_ENTRIES)

GATHER_WINDOW = 128


def _fold8(a):
    p = [a[:, c * 128:(c + 1) * 128] for c in range(GROUPS)]
    return ((p[0] + p[1]) + (p[2] + p[3])) + ((p[4] + p[5]) + (p[6] + p[7]))


def _tc_body(x_ref, w_hbm, b_ref, g_ref, idx_ref, diff_ref,
             w_vmem, wstage_ref, s_ref, t_ref, zmax_ref, zidx_ref,
             sem0, sem1):
    i = pl.program_id(0)

    @pl.when(i == 0)
    def _():
        sems = (sem0, sem1)

        def _chunk_copy(c, sem):
            return pltpu.make_async_copy(
                w_hbm.at[pl.ds(c * BLOCK_M, BLOCK_M), :],
                wstage_ref.at[c % 2], sem)

        _chunk_copy(0, sems[0]).start()
        for c in range(CHUNKS):
            if c + 1 < CHUNKS:
                _chunk_copy(c + 1, sems[(c + 1) % 2]).start()
            _chunk_copy(c, sems[c % 2]).wait()
            w_vmem[pl.ds(c * BLOCK_M, BLOCK_M), :] = (
                wstage_ref[c % 2].astype(jnp.bfloat16))

    xb = x_ref[...].astype(jnp.bfloat16)

    s_ref[...] = jnp.zeros_like(s_ref)
    t_ref[...] = jnp.zeros_like(t_ref)
    zmax_ref[...] = jnp.full_like(zmax_ref, -jnp.inf)
    zidx_ref[...] = jnp.zeros_like(zidx_ref)

    for c in range(CHUNKS):
        l = jax.lax.dot_general(
            xb, w_vmem[pl.ds(c * BLOCK_M, BLOCK_M), :],
            dimension_numbers=(((1,), (1,)), ((), ())),
            preferred_element_type=jnp.float32,
            precision=jax.lax.Precision.DEFAULT,
        ) + b_ref[:, c * BLOCK_M:(c + 1) * BLOCK_M]

        lb = l.astype(jnp.bfloat16)
        eb = jnp.exp(lb)
        s_ref[...] += _fold8(eb).astype(jnp.float32)
        t_ref[...] += _fold8(eb * lb).astype(jnp.float32)

        g = g_ref[:, c * BLOCK_M:(c + 1) * BLOCK_M]
        zmax = zmax_ref[...]
        zidx = zidx_ref[...]
        for q in range(GROUPS):
            zc = l[:, q * 128:(q + 1) * 128] + g[:, q * 128:(q + 1) * 128]
            upd = zc > zmax
            zmax = jnp.where(upd, zc, zmax)
            zidx = jnp.where(upd, jnp.bfloat16(c * GROUPS + q), zidx)
        zmax_ref[...] = zmax
        zidx_ref[...] = zidx

    zm = zmax_ref[...]
    lane = jax.lax.broadcasted_iota(jnp.int32, zm.shape, 1)
    col = zidx_ref[...].astype(jnp.int32) * 128 + lane
    rowmax = jnp.max(zm, axis=1, keepdims=True)
    idx = jnp.min(jnp.where(zm == rowmax, col, jnp.int32(2**30)),
                  axis=1, keepdims=True)
    idx_ref[...] = idx.reshape(1, BLOCK_N)

    s = jnp.sum(s_ref[...], axis=1, keepdims=True)
    t = jnp.sum(t_ref[...], axis=1, keepdims=True)
    drow = t / s - jnp.log(s) + LOGK
    part = (jnp.sum(drow) / N).reshape(1, 1)

    @pl.when(i == 0)
    def _():
        diff_ref[...] = part

    @pl.when(i > 0)
    def _():
        diff_ref[...] = diff_ref[...] + part


@functools.partial(jax.jit, static_argnames=("interpret",))
def _tc_part(x, W, b2d, gumbel, interpret=False):
    return pl.pallas_call(
        _tc_body,
        grid=(GRID_N,),
        in_specs=[
            pl.BlockSpec((BLOCK_N, IN_FEATURES), lambda i: (i, 0)),
            pl.BlockSpec(memory_space=pl.ANY),
            pl.BlockSpec((1, EMBED_ENTRIES), lambda i: (0, 0)),
            pl.BlockSpec((BLOCK_N, EMBED_ENTRIES), lambda i: (i, 0)),
        ],
        out_specs=[
            pl.BlockSpec((1, BLOCK_N), lambda i: (0, i)),
            pl.BlockSpec((1, 1), lambda i: (0, 0)),
        ],
        out_shape=[
            jax.ShapeDtypeStruct((1, N), jnp.int32),
            jax.ShapeDtypeStruct((1, 1), jnp.float32),
        ],
        scratch_shapes=[
            pltpu.VMEM((EMBED_ENTRIES, IN_FEATURES), jnp.bfloat16),
            pltpu.VMEM((2, BLOCK_M, IN_FEATURES), jnp.float32),
            pltpu.VMEM((BLOCK_N, 128), jnp.float32),
            pltpu.VMEM((BLOCK_N, 128), jnp.float32),
            pltpu.VMEM((BLOCK_N, 128), jnp.float32),
            pltpu.VMEM((BLOCK_N, 128), jnp.bfloat16),
            pltpu.SemaphoreType.DMA,
            pltpu.SemaphoreType.DMA,
        ],
        interpret=interpret,
    )(x, W, b2d, gumbel)


def _sc_gather(embed, idx2):
    @functools.partial(
        pl.kernel,
        out_type=jax.ShapeDtypeStruct((N, EMBED_DIM), embed.dtype),
        mesh=plsc.VectorSubcoreMesh(core_axis_name="core",
                                    subcore_axis_name="subcore"),
    )
    def kern(x_hbm, i_hbm, o_hbm):
        def body(i_vmem, o_vmem):
            pltpu.sync_copy(x_hbm.at[i_vmem.at[0]], o_vmem)

        pltpu.emit_pipeline(
            body,
            grid=(N // GATHER_WINDOW,),
            in_specs=[pl.BlockSpec((1, GATHER_WINDOW),
                                   index_map=lambda i: (0, i))],
            out_specs=[pl.BlockSpec((GATHER_WINDOW, EMBED_DIM),
                                    index_map=lambda i: (i, 0))],
            core_axis_name=("core", "subcore"),
            dimension_semantics=(pltpu.PARALLEL,),
        )(i_hbm, o_hbm)

    return kern(embed, idx2)


def kernel(x, W, b, embed, gumbel):
    idx2, diff2d = _tc_part(x, W, b.reshape(1, -1), gumbel)
    quantize = _sc_gather(embed, idx2)
    return (quantize, diff2d.reshape(()), idx2.reshape(N))
